# baseline (speedup 1.0000x reference)
import functools

import jax
import jax.numpy as jnp
from jax import lax
from jax.experimental import pallas as pl
from jax.experimental.pallas import tpu as pltpu
from jax.experimental.pallas import tpu_sc as plsc

B = 16384
D = 32
F = 26
NB = 6


@functools.lru_cache(maxsize=1)
def _build_sc_embed():
    info = plsc.get_sparse_core_info()
    NC, NS = info.num_cores, info.num_subcores
    NW = NC * NS
    BPW = B // NW

    mesh = plsc.VectorSubcoreMesh(core_axis_name="c", subcore_axis_name="s")

    @functools.partial(
        pl.kernel,
        out_type=jax.ShapeDtypeStruct((B, F * D), jnp.float32),
        mesh=mesh,
        compiler_params=pltpu.CompilerParams(use_tc_tiling_on_sc=False),
        scratch_types=[
            pltpu.VMEM((F * BPW,), jnp.int32),
            pltpu.VMEM((NB, BPW, D), jnp.float32),
            pltpu.SemaphoreType.DMA,
            [pltpu.SemaphoreType.DMA] * NB,
            [pltpu.SemaphoreType.DMA] * NB,
        ],
    )
    def sc_embed(*refs):
        feats = refs[0:F]
        tables = refs[F:2 * F]
        out = refs[2 * F]
        idx_v, rows_v, isem, gsems, osems = refs[2 * F + 1:]

        wid = lax.axis_index("s") * NC + lax.axis_index("c")
        base = wid * BPW

        idx_h = [
            pltpu.async_copy(
                feats[f].at[pl.ds(base, BPW)],
                idx_v.at[pl.ds(f * BPW, BPW)], isem)
            for f in range(F)
        ]
        for h in idx_h:
            h.wait()

        gh = [None] * F
        out_h = [None] * F

        def fire_field(f):
            buf = f % NB
            if f >= NB:
                out_h[f - NB].wait()
            gh[f] = pltpu.async_copy(
                tables[f].at[idx_v.at[pl.ds(f * BPW, BPW)]],
                rows_v.at[buf], gsems[buf])

        def retire_field(f):
            buf = f % NB
            gh[f].wait()
            out_h[f] = pltpu.async_copy(
                rows_v.at[buf],
                out.at[pl.ds(base, BPW), pl.ds(f * D, D)], osems[buf])

        LAG = NB - 1
        for f in range(F):
            fire_field(f)
            if f >= LAG:
                retire_field(f - LAG)
        for f in range(F - LAG, F):
            retire_field(f)
        for f in range(F - NB, F):
            out_h[f].wait()

    return sc_embed


def kernel(feat_0, feat_1, feat_2, feat_3, feat_4, feat_5, feat_6, feat_7,
           feat_8, feat_9, feat_10, feat_11, feat_12, feat_13, feat_14,
           feat_15, feat_16, feat_17, feat_18, feat_19, feat_20, feat_21,
           feat_22, feat_23, feat_24, feat_25,
           W_0, W_1, W_2, W_3, W_4, W_5, W_6, W_7,
           W_8, W_9, W_10, W_11, W_12, W_13, W_14, W_15,
           W_16, W_17, W_18, W_19, W_20, W_21, W_22, W_23,
           W_24, W_25):
    feats = [feat_0, feat_1, feat_2, feat_3, feat_4, feat_5, feat_6, feat_7,
             feat_8, feat_9, feat_10, feat_11, feat_12, feat_13, feat_14,
             feat_15, feat_16, feat_17, feat_18, feat_19, feat_20, feat_21,
             feat_22, feat_23, feat_24, feat_25]
    tables = [W_0, W_1, W_2, W_3, W_4, W_5, W_6, W_7,
              W_8, W_9, W_10, W_11, W_12, W_13, W_14, W_15,
              W_16, W_17, W_18, W_19, W_20, W_21, W_22, W_23,
              W_24, W_25]
    return _build_sc_embed()(*feats, *tables)
